# baseline (speedup 1.0000x reference)
import functools

import jax
import jax.numpy as jnp
from jax import lax
from jax.experimental import pallas as pl
from jax.experimental.pallas import tpu as pltpu
from jax.experimental.pallas import tpu_sc as plsc

B, S, D = 4, 4096, 4096
DR = 256
NIN = 2048
K = 512
S_BLK = 128


def _max_reduce_body(x_ref, o_ref):
    s = pl.program_id(0)
    m = jnp.max(x_ref[...], axis=1)

    @pl.when(s == 0)
    def _init():
        o_ref[...] = m

    @pl.when(s != 0)
    def _acc():
        o_ref[...] = jnp.maximum(o_ref[...], m)


def _router_body(gc_ref, w1_ref, b1_ref, g_ref, be_ref, w2_ref, b2_ref,
                 nk_ref, w_ref, rank_ref):
    gc = gc_ref[...]
    h = jnp.dot(gc, w1_ref[...], preferred_element_type=jnp.float32) + b1_ref[...]
    h = 0.5 * h * (1.0 + lax.erf(h * (2.0 ** -0.5)))
    mu = jnp.mean(h, axis=-1, keepdims=True)
    var = jnp.mean((h - mu) ** 2, axis=-1, keepdims=True)
    h = (h - mu) / jnp.sqrt(var + 1e-5) * g_ref[...] + be_ref[...]
    q = jnp.dot(h, w2_ref[...], preferred_element_type=jnp.float32) + b2_ref[...]
    nk = nk_ref[...]
    scale = 1.0 / (DR ** 0.5)
    logits = lax.dot_general(q, nk, (((1,), (1,)), ((), ())),
                             preferred_element_type=jnp.float32) * scale
    eye = (lax.broadcasted_iota(jnp.int32, (NIN, NIN), 0)
           == lax.broadcasted_iota(jnp.int32, (NIN, NIN), 1)).astype(jnp.float32)
    logits_t = lax.dot_general(eye, logits, (((0,), (1,)), ((), ())),
                               precision=lax.Precision.HIGHEST,
                               preferred_element_type=jnp.float32)
    z = logits * 10.0
    zmax = jnp.max(z, axis=-1, keepdims=True)
    ez = jnp.exp(z - zmax)
    p = ez / jnp.sum(ez, axis=-1, keepdims=True)

    CH = 256
    n_ch = NIN // CH
    j_idx = lax.broadcasted_iota(jnp.int32, (CH, NIN), 1)
    i_lt_j = []
    for c in range(n_ch):
        i_idx = lax.broadcasted_iota(jnp.int32, (CH, NIN), 0) + c * CH
        i_lt_j.append(i_idx < j_idx)
    for b in range(B):
        rowv = logits[b:b + 1, :]
        acc = jnp.zeros((1, NIN), jnp.int32)
        for c in range(n_ch):
            colv = logits_t[c * CH:(c + 1) * CH, b:b + 1]
            gt = colv > rowv
            before = gt | ((colv == rowv) & i_lt_j[c])
            acc = acc + jnp.sum(before.astype(jnp.int32), axis=0, keepdims=True)
        rank_ref[b:b + 1, :] = acc
        maskf = (acc < K).astype(jnp.float32)
        pb = p[b:b + 1, :]
        w_ref[b:b + 1, :] = (maskf + pb) - pb


@functools.lru_cache(maxsize=1)
def _make_sc_scatter():
    info = plsc.get_sparse_core_info()
    nc = info.num_cores

    @functools.partial(
        pl.kernel,
        mesh=plsc.VectorSubcoreMesh(core_axis_name="c", subcore_axis_name="s"),
        compiler_params=pltpu.CompilerParams(needs_layout_passes=False),
        out_type=jax.ShapeDtypeStruct((B, K), jnp.int32),
        scratch_types=[
            pltpu.VMEM((NIN,), jnp.int32),
            pltpu.VMEM((K,), jnp.int32),
        ],
    )
    def sc_scatter(rank_hbm, out_hbm, rank_v, out_v):
        wid = lax.axis_index("s") * nc + lax.axis_index("c")

        @pl.when(wid < B)
        def _():
            pltpu.sync_copy(rank_hbm.at[wid], rank_v)

            def body(c, carry):
                idx = rank_v[pl.ds(c * 16, 16)]
                vals = c * 16 + lax.iota(jnp.int32, 16)
                m = idx < K
                safe_idx = jnp.where(m, idx, 0)
                plsc.store_scatter(out_v, [safe_idx], vals, mask=m)
                return carry

            lax.fori_loop(0, NIN // 16, body, 0)
            pltpu.sync_copy(out_v, out_hbm.at[wid])

    return sc_scatter


def kernel(x, W1, b1, ln_g, ln_b, W2, b2, neuron_keys, k_input):
    del k_input
    gc = pl.pallas_call(
        _max_reduce_body,
        grid=(S // S_BLK,),
        in_specs=[pl.BlockSpec((B, S_BLK, D), lambda s: (0, s, 0))],
        out_specs=pl.BlockSpec((B, D), lambda s: (0, 0)),
        out_shape=jax.ShapeDtypeStruct((B, D), jnp.float32),
    )(x)

    weights, rank = pl.pallas_call(
        _router_body,
        in_specs=[
            pl.BlockSpec((B, D), lambda: (0, 0)),
            pl.BlockSpec((D, 2 * DR), lambda: (0, 0)),
            pl.BlockSpec((1, 2 * DR), lambda: (0, 0)),
            pl.BlockSpec((1, 2 * DR), lambda: (0, 0)),
            pl.BlockSpec((1, 2 * DR), lambda: (0, 0)),
            pl.BlockSpec((2 * DR, DR), lambda: (0, 0)),
            pl.BlockSpec((1, DR), lambda: (0, 0)),
            pl.BlockSpec((NIN, DR), lambda: (0, 0)),
        ],
        out_specs=[
            pl.BlockSpec((B, NIN), lambda: (0, 0)),
            pl.BlockSpec((B, NIN), lambda: (0, 0)),
        ],
        out_shape=[
            jax.ShapeDtypeStruct((B, NIN), jnp.float32),
            jax.ShapeDtypeStruct((B, NIN), jnp.int32),
        ],
    )(gc, W1, b1.reshape(1, -1), ln_g.reshape(1, -1), ln_b.reshape(1, -1),
      W2, b2.reshape(1, -1), neuron_keys)

    input_idx = _make_sc_scatter()(rank)
    return input_idx, weights


# speedup vs baseline: 1.1231x; 1.1231x over previous
import functools

import jax
import jax.numpy as jnp
from jax import lax
from jax.experimental import pallas as pl
from jax.experimental.pallas import tpu as pltpu
from jax.experimental.pallas import tpu_sc as plsc

B, S, D = 4, 4096, 4096
DR = 256
NIN = 2048
K = 512
S_BLK = 128
N_STEPS = S // S_BLK


def _fused_body(x_ref, w1_ref, b1_ref, g_ref, be_ref, w2_ref, b2_ref,
                nk_ref, w_ref, rank_ref, gc_ref):
    s = pl.program_id(0)
    m = jnp.max(x_ref[...], axis=1)

    @pl.when(s == 0)
    def _init():
        gc_ref[...] = m

    @pl.when(s != 0)
    def _acc():
        gc_ref[...] = jnp.maximum(gc_ref[...], m)

    @pl.when(s == N_STEPS - 1)
    def _head():
        gc = gc_ref[...]
        h = jnp.dot(gc, w1_ref[...],
                    preferred_element_type=jnp.float32) + b1_ref[...]
        h = 0.5 * h * (1.0 + lax.erf(h * (2.0 ** -0.5)))
        mu = jnp.mean(h, axis=-1, keepdims=True)
        var = jnp.mean((h - mu) ** 2, axis=-1, keepdims=True)
        h = (h - mu) / jnp.sqrt(var + 1e-5) * g_ref[...] + be_ref[...]
        q = jnp.dot(h, w2_ref[...],
                    preferred_element_type=jnp.float32) + b2_ref[...]
        nk = nk_ref[...]
        scale = 1.0 / (DR ** 0.5)
        logits = lax.dot_general(q, nk, (((1,), (1,)), ((), ())),
                                 preferred_element_type=jnp.float32) * scale
        logits_t = jnp.transpose(logits, (1, 0))
        z = logits * 10.0
        zmax = jnp.max(z, axis=-1, keepdims=True)
        ez = jnp.exp(z - zmax)
        p = ez / jnp.sum(ez, axis=-1, keepdims=True)

        CH = 256
        n_ch = NIN // CH
        j_idx = lax.broadcasted_iota(jnp.int32, (CH, NIN), 1)
        i_lt_j = []
        for c in range(n_ch):
            i_idx = lax.broadcasted_iota(jnp.int32, (CH, NIN), 0) + c * CH
            i_lt_j.append(i_idx < j_idx)
        for b in range(B):
            rowv = logits[b:b + 1, :]
            acc = jnp.zeros((1, NIN), jnp.int32)
            for c in range(n_ch):
                colv = logits_t[c * CH:(c + 1) * CH, b:b + 1]
                gt = colv > rowv
                before = gt | ((colv == rowv) & i_lt_j[c])
                acc = acc + jnp.sum(before.astype(jnp.int32), axis=0,
                                    keepdims=True)
            rank_ref[b:b + 1, :] = acc
            maskf = (acc < K).astype(jnp.float32)
            pb = p[b:b + 1, :]
            w_ref[b:b + 1, :] = (maskf + pb) - pb


@functools.lru_cache(maxsize=1)
def _make_sc_scatter():
    info = plsc.get_sparse_core_info()
    nc = info.num_cores

    @functools.partial(
        pl.kernel,
        mesh=plsc.VectorSubcoreMesh(core_axis_name="c", subcore_axis_name="s"),
        compiler_params=pltpu.CompilerParams(needs_layout_passes=False),
        out_type=jax.ShapeDtypeStruct((B, K), jnp.int32),
        scratch_types=[
            pltpu.VMEM((NIN,), jnp.int32),
            pltpu.VMEM((K,), jnp.int32),
        ],
    )
    def sc_scatter(rank_hbm, out_hbm, rank_v, out_v):
        wid = lax.axis_index("s") * nc + lax.axis_index("c")

        @pl.when(wid < B)
        def _():
            pltpu.sync_copy(rank_hbm.at[wid], rank_v)

            def body(c, carry):
                idx = rank_v[pl.ds(c * 16, 16)]
                vals = c * 16 + lax.iota(jnp.int32, 16)
                m = idx < K
                safe_idx = jnp.where(m, idx, 0)
                plsc.store_scatter(out_v, [safe_idx], vals, mask=m)
                return carry

            lax.fori_loop(0, NIN // 16, body, 0)
            pltpu.sync_copy(out_v, out_hbm.at[wid])

    return sc_scatter


def kernel(x, W1, b1, ln_g, ln_b, W2, b2, neuron_keys, k_input):
    del k_input
    c0 = lambda s: (0, 0)
    weights, rank = pl.pallas_call(
        _fused_body,
        grid=(N_STEPS,),
        in_specs=[
            pl.BlockSpec((B, S_BLK, D), lambda s: (0, s, 0)),
            pl.BlockSpec((D, 2 * DR), c0),
            pl.BlockSpec((1, 2 * DR), c0),
            pl.BlockSpec((1, 2 * DR), c0),
            pl.BlockSpec((1, 2 * DR), c0),
            pl.BlockSpec((2 * DR, DR), c0),
            pl.BlockSpec((1, DR), c0),
            pl.BlockSpec((NIN, DR), c0),
        ],
        out_specs=[
            pl.BlockSpec((B, NIN), c0),
            pl.BlockSpec((B, NIN), c0),
        ],
        out_shape=[
            jax.ShapeDtypeStruct((B, NIN), jnp.float32),
            jax.ShapeDtypeStruct((B, NIN), jnp.int32),
        ],
        scratch_shapes=[pltpu.VMEM((B, D), jnp.float32)],
    )(x, W1, b1.reshape(1, -1), ln_g.reshape(1, -1), ln_b.reshape(1, -1),
      W2, b2.reshape(1, -1), neuron_keys)

    input_idx = _make_sc_scatter()(rank)
    return input_idx, weights
